# Initial kernel scaffold; baseline (speedup 1.0000x reference)
import jax
import jax.numpy as jnp
from jax.experimental import pallas as pl


def kernel(x, mask, W_e, b_e, W1, b1, W2, b2, codebook):
    raise NotImplementedError("write your pallas kernel here")


# speedup vs baseline: 1.2205x; 1.2205x over previous
import jax
import jax.numpy as jnp
from jax.experimental import pallas as pl
from jax.experimental.pallas import tpu as pltpu

B, C, H, W_IMG = 16, 3, 384, 384
P = 16
PATCH_DIM = C * P * P
HID = 768
FF = 1536
D = 256
K = 8192
N_TOK = (H // P) * (W_IMG // P)
M = B * N_TOK

M_TILE = 512
K_TILE = 2048


_PREC = jax.lax.Precision.DEFAULT


def _vq_body(p_ref, we_ref, be_ref, w1_ref, b1_ref, w2_ref, b2_ref,
             cbt_ref, out_ref):
    h = jnp.dot(p_ref[...], we_ref[...], preferred_element_type=jnp.float32,
                precision=_PREC)
    h = jax.nn.gelu(h + be_ref[...])
    h = jnp.dot(h, w1_ref[...], preferred_element_type=jnp.float32,
                precision=_PREC)
    h = jax.nn.gelu(h + b1_ref[...])
    hf = jnp.dot(h, w2_ref[...], preferred_element_type=jnp.float32,
                 precision=_PREC)
    hf = hf + b2_ref[...]
    h2 = jnp.sum(hf * hf, axis=1, keepdims=True)

    bmin = jnp.full((M_TILE,), jnp.inf, dtype=jnp.float32)
    barg = jnp.zeros((M_TILE,), dtype=jnp.int32)
    for kt in range(K // K_TILE):
        cbt = cbt_ref[:, kt * K_TILE:(kt + 1) * K_TILE]
        c2 = jnp.sum(cbt * cbt, axis=0, keepdims=True)
        s = jnp.dot(hf, cbt, preferred_element_type=jnp.float32,
                    precision=_PREC)
        d2 = (h2 - 2.0 * s) + c2
        tmin = jnp.min(d2, axis=1)
        targ = jnp.argmin(d2, axis=1).astype(jnp.int32) + kt * K_TILE
        upd = tmin < bmin
        barg = jnp.where(upd, targ, barg)
        bmin = jnp.where(upd, tmin, bmin)
    out_ref[...] = barg


def _patchify(x):
    b, c, h, w = x.shape
    x = x.reshape(b, c, h // P, P, w // P, P)
    x = x.transpose(0, 2, 4, 1, 3, 5)
    return x.reshape(b * (h // P) * (w // P), c * P * P)


def kernel(x, mask, W_e, b_e, W1, b1, W2, b2, codebook):
    patches = _patchify(x)
    cbt = codebook.T

    tokens = pl.pallas_call(
        _vq_body,
        grid=(M // M_TILE,),
        in_specs=[
            pl.BlockSpec((M_TILE, PATCH_DIM), lambda i: (i, 0)),
            pl.BlockSpec((PATCH_DIM, HID), lambda i: (0, 0)),
            pl.BlockSpec((1, HID), lambda i: (0, 0)),
            pl.BlockSpec((HID, FF), lambda i: (0, 0)),
            pl.BlockSpec((1, FF), lambda i: (0, 0)),
            pl.BlockSpec((FF, D), lambda i: (0, 0)),
            pl.BlockSpec((1, D), lambda i: (0, 0)),
            pl.BlockSpec((D, K), lambda i: (0, 0)),
        ],
        out_specs=pl.BlockSpec((M_TILE,), lambda i: (i,)),
        out_shape=jax.ShapeDtypeStruct((M,), jnp.int32),
        compiler_params=pltpu.CompilerParams(
            dimension_semantics=("parallel",),
        ),
    )(patches, W_e, b_e.reshape(1, HID), W1, b1.reshape(1, FF),
      W2, b2.reshape(1, D), cbt)

    tokens = tokens.reshape(B, N_TOK)
    return jnp.where(mask, tokens, -1)


# speedup vs baseline: 1.7962x; 1.4716x over previous
import jax
import jax.numpy as jnp
from jax.experimental import pallas as pl
from jax.experimental.pallas import tpu as pltpu

B, C, H, W_IMG = 16, 3, 384, 384
P = 16
G = H // P
PATCH_DIM = C * P * P
HID = 768
FF = 1536
D = 256
K = 8192
N_TOK = G * G
M = B * N_TOK

K_TILE = 2048


def _c2_body(cb_ref, c2_ref):
    cb = cb_ref[...]
    c2_ref[...] = jnp.sum(cb * cb, axis=1).reshape(1, K)


def _vq_body(x_ref, we_ref, be_ref, w1_ref, b1_ref, w2_ref, b2_ref,
             cb_ref, c2_ref, out_ref):
    xb = x_ref[0]
    patches = jnp.transpose(xb, (1, 3, 0, 2, 4)).reshape(N_TOK, PATCH_DIM)

    h = jnp.dot(patches, we_ref[...], preferred_element_type=jnp.float32)
    h = jax.nn.gelu(h + be_ref[...])
    h = jnp.dot(h, w1_ref[...], preferred_element_type=jnp.float32)
    h = jax.nn.gelu(h + b1_ref[...])
    hf = jnp.dot(h, w2_ref[...], preferred_element_type=jnp.float32)
    hf = hf + b2_ref[...]
    h2 = jnp.sum(hf * hf, axis=1, keepdims=True)
    hfn2 = -2.0 * hf

    bmin = jnp.full((N_TOK,), jnp.inf, dtype=jnp.float32)
    barg = jnp.zeros((N_TOK,), dtype=jnp.int32)
    for kt in range(K // K_TILE):
        cb = cb_ref[kt * K_TILE:(kt + 1) * K_TILE, :]
        s2 = jax.lax.dot_general(hfn2, cb, (((1,), (1,)), ((), ())),
                                 preferred_element_type=jnp.float32)
        d2 = (h2 + s2) + c2_ref[:, kt * K_TILE:(kt + 1) * K_TILE]
        tmin = jnp.min(d2, axis=1)
        targ = jnp.argmin(d2, axis=1).astype(jnp.int32) + kt * K_TILE
        upd = tmin < bmin
        barg = jnp.where(upd, targ, barg)
        bmin = jnp.where(upd, tmin, bmin)
    out_ref[...] = barg.reshape(1, 1, N_TOK)


def kernel(x, mask, W_e, b_e, W1, b1, W2, b2, codebook):
    x6 = x.reshape(B, C, G, P, G, P)

    c2 = pl.pallas_call(
        _c2_body,
        in_specs=[pl.BlockSpec((K, D), lambda: (0, 0))],
        out_specs=pl.BlockSpec((1, K), lambda: (0, 0)),
        out_shape=jax.ShapeDtypeStruct((1, K), jnp.float32),
    )(codebook)

    tokens = pl.pallas_call(
        _vq_body,
        grid=(B,),
        in_specs=[
            pl.BlockSpec((1, C, G, P, G, P), lambda i: (i, 0, 0, 0, 0, 0)),
            pl.BlockSpec((PATCH_DIM, HID), lambda i: (0, 0)),
            pl.BlockSpec((1, HID), lambda i: (0, 0)),
            pl.BlockSpec((HID, FF), lambda i: (0, 0)),
            pl.BlockSpec((1, FF), lambda i: (0, 0)),
            pl.BlockSpec((FF, D), lambda i: (0, 0)),
            pl.BlockSpec((1, D), lambda i: (0, 0)),
            pl.BlockSpec((K, D), lambda i: (0, 0)),
            pl.BlockSpec((1, K), lambda i: (0, 0)),
        ],
        out_specs=pl.BlockSpec((1, 1, N_TOK), lambda i: (i, 0, 0)),
        out_shape=jax.ShapeDtypeStruct((B, 1, N_TOK), jnp.int32),
        compiler_params=pltpu.CompilerParams(
            dimension_semantics=("parallel",),
        ),
    )(x6, W_e, b_e.reshape(1, HID), W1, b1.reshape(1, FF),
      W2, b2.reshape(1, D), codebook, c2)

    tokens = tokens.reshape(B, N_TOK)
    return jnp.where(mask, tokens, -1)
